# baseline (speedup 1.0000x reference)
import jax
import jax.numpy as jnp
from jax import lax
from jax.experimental import pallas as pl
from jax.experimental.pallas import tpu as pltpu
from jax.experimental.pallas import tpu_sc as plsc

VOCAB = 1000
VPAD = 1024
EMB = 300
NCLS = 20
B = 4096
L = 50

NC, NS = 2, 16
NW = NC * NS
DOCS_PER_W = B // NW
GROUPS = DOCS_PER_W // 16


def _tc_project(table_ref, w_ref, b_ref, mt_ref, bias_ref):
    mt = lax.dot_general(
        w_ref[...], table_ref[...],
        (((1,), (1,)), ((), ())),
        preferred_element_type=jnp.float32,
    ) * (1.0 / L)
    mt_ref[:, :VOCAB] = mt
    mt_ref[:, VOCAB:] = jnp.zeros((NCLS, VPAD - VOCAB), jnp.float32)
    bias_ref[...] = jnp.broadcast_to(b_ref[...], (NCLS, 16))


def _sc_pool(mt_hbm, bias_hbm, docs_hbm, out_hbm, mt_v, bias_v, docs_v, out_v):
    cid = lax.axis_index("c")
    sid = lax.axis_index("s")
    wid = sid * NC + cid
    row0 = wid * DOCS_PER_W
    pltpu.sync_copy(mt_hbm, mt_v)
    pltpu.sync_copy(bias_hbm, bias_v)
    pltpu.sync_copy(docs_hbm.at[pl.ds(row0, DOCS_PER_W), :], docs_v)
    lane = lax.iota(jnp.int32, 16)
    cols = [jnp.full((16,), c, jnp.int32) for c in range(NCLS)]
    for g in range(GROUPS):
        doc = lane + g * 16
        acc0 = tuple(bias_v[c, :] for c in range(NCLS))

        def step(l, accs, doc=doc):
            tok = plsc.load_gather(docs_v, [doc, jnp.broadcast_to(l, (16,))])
            return tuple(accs[c] + plsc.load_gather(mt_v, [cols[c], tok])
                         for c in range(NCLS))

        accs = lax.fori_loop(0, L, step, acc0)
        for c in range(NCLS):
            plsc.store_scatter(out_v, [doc, cols[c]], accs[c])
    pltpu.sync_copy(out_v, out_hbm.at[pl.ds(row0, DOCS_PER_W), :])


def kernel(docs, table, W, b):
    mt, bias_b = pl.pallas_call(
        _tc_project,
        out_shape=(
            jax.ShapeDtypeStruct((NCLS, VPAD), jnp.float32),
            jax.ShapeDtypeStruct((NCLS, 16), jnp.float32),
        ),
    )(table, W, b.reshape(NCLS, 1))

    mesh = plsc.VectorSubcoreMesh(core_axis_name="c", subcore_axis_name="s",
                                  num_cores=NC, num_subcores=NS)
    sc = pl.kernel(
        _sc_pool,
        out_type=jax.ShapeDtypeStruct((B, NCLS), jnp.float32),
        mesh=mesh,
        compiler_params=pltpu.CompilerParams(needs_layout_passes=False,
                                             use_tc_tiling_on_sc=True),
        scratch_types=[
            pltpu.VMEM((NCLS, VPAD), jnp.float32),
            pltpu.VMEM((NCLS, 16), jnp.float32),
            pltpu.VMEM((DOCS_PER_W, L), jnp.int32),
            pltpu.VMEM((DOCS_PER_W, NCLS), jnp.float32),
        ],
    )
    return sc(mt, bias_b, docs)


# speedup vs baseline: 1.0554x; 1.0554x over previous
import jax
import jax.numpy as jnp
from jax import lax
from jax.experimental import pallas as pl
from jax.experimental.pallas import tpu as pltpu
from jax.experimental.pallas import tpu_sc as plsc

VOCAB = 1000
VPAD = 1024
EMB = 300
NCLS = 20
B = 4096
L = 50

NC, NS = 2, 16
NW = NC * NS
DOCS_PER_W = B // NW
GROUPS = DOCS_PER_W // 16


def _tc_project(tablet_ref, w_ref, b_ref, mt_ref, bias_ref):
    mt = lax.dot_general(
        w_ref[...], tablet_ref[...],
        (((1,), (0,)), ((), ())),
        preferred_element_type=jnp.float32,
    ) * (1.0 / L)
    mt_ref[:, :VOCAB] = mt
    mt_ref[:, VOCAB:] = jnp.zeros((NCLS, VPAD - VOCAB), jnp.float32)
    bias_ref[...] = jnp.broadcast_to(b_ref[...], (NCLS, 16))


def _sc_pool(mt_hbm, bias_hbm, docst_hbm, outt_hbm, m_v, bias_v, docs_v, out_v):
    cid = lax.axis_index("c")
    sid = lax.axis_index("s")
    wid = sid * NC + cid
    col0 = wid * DOCS_PER_W
    for c in range(NCLS):
        pltpu.sync_copy(mt_hbm.at[c], m_v.at[pl.ds(c * VPAD, VPAD)])
    pltpu.sync_copy(bias_hbm, bias_v)
    pltpu.sync_copy(docst_hbm.at[:, pl.ds(col0, DOCS_PER_W)], docs_v)
    for g in range(GROUPS):
        acc0 = tuple(bias_v[c, :] for c in range(NCLS))

        def step(l, accs, g=g):
            tok = docs_v[l, pl.ds(g * 16, 16)]
            return tuple(accs[c] + plsc.load_gather(m_v, [tok + c * VPAD])
                         for c in range(NCLS))

        accs = lax.fori_loop(0, L, step, acc0)
        for c in range(NCLS):
            out_v[c, pl.ds(g * 16, 16)] = accs[c]
    pltpu.sync_copy(out_v, outt_hbm.at[:, pl.ds(col0, DOCS_PER_W)])


def kernel(docs, table, W, b):
    mt, bias_b = pl.pallas_call(
        _tc_project,
        out_shape=(
            jax.ShapeDtypeStruct((NCLS, VPAD), jnp.float32),
            jax.ShapeDtypeStruct((NCLS, 16), jnp.float32),
        ),
    )(table.T, W, b.reshape(NCLS, 1))

    mesh = plsc.VectorSubcoreMesh(core_axis_name="c", subcore_axis_name="s",
                                  num_cores=NC, num_subcores=NS)
    sc = pl.kernel(
        _sc_pool,
        out_type=jax.ShapeDtypeStruct((NCLS, B), jnp.float32),
        mesh=mesh,
        compiler_params=pltpu.CompilerParams(needs_layout_passes=False),
        scratch_types=[
            pltpu.VMEM((NCLS * VPAD,), jnp.float32),
            pltpu.VMEM((NCLS, 16), jnp.float32),
            pltpu.VMEM((L, DOCS_PER_W), jnp.int32),
            pltpu.VMEM((NCLS, DOCS_PER_W), jnp.float32),
        ],
    )
    out_t = sc(mt, bias_b, docs.T)
    return out_t.T


# speedup vs baseline: 1.4295x; 1.3544x over previous
import jax
import jax.numpy as jnp
from jax import lax
from jax.experimental import pallas as pl
from jax.experimental.pallas import tpu as pltpu
from jax.experimental.pallas import tpu_sc as plsc

VOCAB = 1000
VPAD = 1024
EMB = 300
NCLS = 20
B = 4096
L = 50

NC, NS = 2, 16
NW = NC * NS
DOCS_PER_W = B // NW
GROUPS = DOCS_PER_W // 16


def _tc_project(tablet_ref, w_ref, mt_ref):
    mt = lax.dot_general(
        w_ref[...], tablet_ref[...],
        (((1,), (0,)), ((), ())),
        preferred_element_type=jnp.float32,
    ) * (1.0 / L)
    mt_ref[:, :VOCAB] = mt
    mt_ref[:, VOCAB:] = jnp.zeros((NCLS, VPAD - VOCAB), jnp.float32)


def _sc_pool(mt_hbm, b_hbm, docst_hbm, outt_hbm, m_v, bias_v, docs_v, out_v,
             sem):
    cid = lax.axis_index("c")
    sid = lax.axis_index("s")
    wid = sid * NC + cid
    col0 = wid * DOCS_PER_W
    cps = [pltpu.async_copy(docst_hbm.at[:, pl.ds(col0, DOCS_PER_W)], docs_v,
                            sem),
           pltpu.async_copy(b_hbm, bias_v, sem)]
    cps += [pltpu.async_copy(mt_hbm.at[c], m_v.at[pl.ds(c * VPAD, VPAD)], sem)
            for c in range(NCLS)]
    for cp in cps:
        cp.wait()
    cls_idx = [jnp.full((16,), c, jnp.int32) for c in range(NCLS)]
    for g in range(GROUPS):
        acc0 = tuple(plsc.load_gather(bias_v, [cls_idx[c]])
                     for c in range(NCLS))

        def step(l, accs, g=g):
            tok = docs_v[l, pl.ds(g * 16, 16)]
            return tuple(accs[c] + plsc.load_gather(m_v, [tok + c * VPAD])
                         for c in range(NCLS))

        accs = lax.fori_loop(0, L, step, acc0)
        for c in range(NCLS):
            out_v[c, pl.ds(g * 16, 16)] = accs[c]
    pltpu.sync_copy(out_v, outt_hbm.at[:, pl.ds(col0, DOCS_PER_W)])


def kernel(docs, table, W, b):
    mt = pl.pallas_call(
        _tc_project,
        out_shape=jax.ShapeDtypeStruct((NCLS, VPAD), jnp.float32),
    )(table.T, W)

    mesh = plsc.VectorSubcoreMesh(core_axis_name="c", subcore_axis_name="s",
                                  num_cores=NC, num_subcores=NS)
    sc = pl.kernel(
        _sc_pool,
        out_type=jax.ShapeDtypeStruct((NCLS, B), jnp.float32),
        mesh=mesh,
        compiler_params=pltpu.CompilerParams(needs_layout_passes=False),
        scratch_types=[
            pltpu.VMEM((NCLS * VPAD,), jnp.float32),
            pltpu.VMEM((NCLS,), jnp.float32),
            pltpu.VMEM((L, DOCS_PER_W), jnp.int32),
            pltpu.VMEM((NCLS, DOCS_PER_W), jnp.float32),
            pltpu.SemaphoreType.DMA,
        ],
    )
    out_t = sc(mt, b, docs.T)
    return out_t.T
